# baseline (speedup 1.0000x reference)
import jax
import jax.numpy as jnp
from jax import lax
from jax.experimental import pallas as pl
from jax.experimental.pallas import tpu as pltpu
from jax.experimental.pallas import tpu_sc as plsc

B = 16384
D = 64
K = 5
E = 1000000
NC = 2
NS = 16
L = 16
NW = NC * NS
PER_W = B // NW
C = 128
CHUNKS = PER_W // C
GROUPS = C // L

BLK = 24576
NSTEP = (E + BLK - 1) // BLK
NPACK = NSTEP * BLK


def _pack_body(u_ref, v_ref, out_ref):
    z = jnp.concatenate([u_ref[...], v_ref[...]], axis=0)
    out_ref[...] = z.T


_pack = pl.pallas_call(
    _pack_body,
    grid=(NSTEP,),
    in_specs=[pl.BlockSpec((D, BLK), lambda i: (0, i)),
              pl.BlockSpec((D, BLK), lambda i: (0, i))],
    out_specs=pl.BlockSpec((BLK, 2 * D), lambda i: (i, 0)),
    out_shape=jax.ShapeDtypeStruct((NPACK, 2 * D), jnp.float32),
)


def _sc_body(tbl_hbm, pos_u_hbm, pos_v_hbm, neg_hbm,
             pos_out_hbm, neg_out_hbm,
             idx_uv, idx_n, rows_uv, rows_n,
             outp, outn, s_i0, s_i1, s_i2, s_i3, s_r0, s_r1, s_o):
    wid = lax.axis_index("s") * NC + lax.axis_index("c")
    iota = lax.iota(jnp.int32, L)
    sem_i = [s_i0, s_i1, s_i2, s_i3]
    sem_r = [s_r0, s_r1]
    base = wid * PER_W

    idx_cp = []
    for c in range(CHUNKS):
        b0 = base + c * C
        idx_cp.append([
            pltpu.async_copy(pos_u_hbm.at[pl.ds(b0, C)],
                             idx_uv.at[c, pl.ds(0, C)], sem_i[c]),
            pltpu.async_copy(pos_v_hbm.at[pl.ds(b0, C)],
                             idx_uv.at[c, pl.ds(C, C)], sem_i[c]),
            pltpu.async_copy(neg_hbm.at[pl.ds(b0 * K, C * K)], idx_n.at[c],
                             sem_i[c]),
        ])

    def transform_and_fire(c):
        for cp in idx_cp[c]:
            cp.wait()
        buf = c % 2
        for g in range(2 * GROUPS):
            off = 0 if g < GROUPS else 1
            vals = plsc.load_gather(idx_uv.at[c], [g * L + iota])
            idx_uv[c, pl.ds(g * L, L)] = vals * 2 + off
        for j in range(K * C // L):
            vals = plsc.load_gather(idx_n.at[c], [j * L + iota])
            idx_n[c, pl.ds(j * L, L)] = vals * 2 + 1
        return [pltpu.async_copy(tbl_hbm.at[idx_uv.at[c]], rows_uv.at[buf],
                                 sem_r[buf]),
                pltpu.async_copy(tbl_hbm.at[idx_n.at[c]], rows_n.at[buf],
                                 sem_r[buf])]

    pending = transform_and_fire(0)
    out_cp = []
    for c in range(CHUNKS):
        buf = c % 2
        cur = pending
        if c + 1 < CHUNKS:
            pending = transform_and_fire(c + 1)
        for cp in cur:
            cp.wait()
        ruv = rows_uv.at[buf]
        rn = rows_n.at[buf]

        def group_body(g, carry2):
            bvec = g * L + iota
            bvK = bvec * K

            def d_body(d, acc):
                dvec = jnp.full((L,), d, jnp.int32)
                u_d = plsc.load_gather(ruv, [bvec, dvec])
                v_d = plsc.load_gather(ruv, [bvec + C, dvec])
                new = [acc[0] + u_d * v_d]
                for k in range(K):
                    n_d = plsc.load_gather(rn, [bvK + k, dvec])
                    new.append(acc[k + 1] + n_d * u_d)
                return tuple(new)

            z = jnp.zeros((L,), jnp.float32)
            acc = lax.fori_loop(0, D, d_body, (z,) * (K + 1))
            outp[c, pl.ds(g * L, L)] = acc[0]
            for k in range(K):
                outn[c, k, pl.ds(g * L, L)] = acc[k + 1]
            return carry2

        lax.fori_loop(0, GROUPS, group_body, 0)
        b0 = base + c * C
        out_cp.append(pltpu.async_copy(outp.at[c], pos_out_hbm.at[pl.ds(b0, C)],
                                       s_o))
        for k in range(K):
            out_cp.append(pltpu.async_copy(
                outn.at[c, k], neg_out_hbm.at[pl.ds(k * B + b0, C)], s_o))
    for cp in out_cp:
        cp.wait()


_sc_dots = pl.kernel(
    _sc_body,
    out_type=[jax.ShapeDtypeStruct((B,), jnp.float32),
              jax.ShapeDtypeStruct((K * B,), jnp.float32)],
    mesh=plsc.VectorSubcoreMesh(core_axis_name="c", subcore_axis_name="s",
                                num_cores=NC, num_subcores=NS),
    compiler_params=pltpu.CompilerParams(needs_layout_passes=False,
                                         use_tc_tiling_on_sc=False),
    scratch_types=[
        pltpu.VMEM((CHUNKS, 2 * C), jnp.int32),
        pltpu.VMEM((CHUNKS, C * K), jnp.int32),
        pltpu.VMEM((2, 2 * C, D), jnp.float32),
        pltpu.VMEM((2, K * C, D), jnp.float32),
        pltpu.VMEM((CHUNKS, C), jnp.float32),
        pltpu.VMEM((CHUNKS, K, C), jnp.float32),
        pltpu.SemaphoreType.DMA,
        pltpu.SemaphoreType.DMA,
        pltpu.SemaphoreType.DMA,
        pltpu.SemaphoreType.DMA,
        pltpu.SemaphoreType.DMA,
        pltpu.SemaphoreType.DMA,
        pltpu.SemaphoreType.DMA,
    ],
)


def _tc_body(pos_ref, neg_ref, out_ref):
    p = jnp.clip(pos_ref[...], -10.0, 10.0)
    n = jnp.clip(neg_ref[...], -10.0, 10.0)
    tot = jnp.sum(jnp.log1p(jnp.exp(-p))) + jnp.sum(jnp.log1p(jnp.exp(n)))
    out_ref[0, 0] = tot * jnp.float32(1.0 / B)


_tc_finish = pl.pallas_call(
    _tc_body,
    out_shape=jax.ShapeDtypeStruct((1, 1), jnp.float32),
    out_specs=pl.BlockSpec(memory_space=pltpu.SMEM),
)


def kernel(u_embeddings, v_embeddings, pos_u, pos_v, neg_v):
    ut = u_embeddings.T
    vt = v_embeddings.T
    packed = _pack(ut, vt)
    neg_flat = neg_v.reshape(B * K)
    pos_dots, neg_dots = _sc_dots(packed.reshape(2 * NPACK, D),
                                  pos_u, pos_v, neg_flat)
    res = _tc_finish(pos_dots.reshape(B // 128, 128),
                     neg_dots.reshape(K * B // 128, 128))
    return res[0, 0]
